# Initial kernel scaffold; baseline (speedup 1.0000x reference)
import jax
import jax.numpy as jnp
from jax.experimental import pallas as pl


def kernel(Z, hyperedge_index, num_nodes):
    raise NotImplementedError("write your pallas kernel here")


# speedup vs baseline: 18.8035x; 18.8035x over previous
import functools

import jax
import jax.numpy as jnp
from jax import lax
from jax.experimental import pallas as pl
from jax.experimental.pallas import tpu as pltpu
from jax.experimental.pallas import tpu_sc as plsc

NC = 2
NS = 16
NW = NC * NS

_N = 100000
_K = 8
_E = 3200000
_R = _E // 128
_G = 8
_NGRP = _R // _G
_NH = 102400
_HS = _NH // NS
_NP = 100096
_SR = _NP // NS

_mesh = plsc.VectorSubcoreMesh(
    core_axis_name="c", subcore_axis_name="s", num_cores=NC, num_subcores=NS
)


def _grp_range(wid):
  g0 = (_NGRP * wid) // NW
  g1 = (_NGRP * (wid + 1)) // NW
  return g0, g1


def _hist_body(nidx, eidx, out, hv, he, zbuf, onesb, nbuf, ebuf):
  c = lax.axis_index("c")
  s = lax.axis_index("s")
  wid = c * NS + s

  def zero_zbuf(i, carry):
    zbuf[pl.ds(i * 16, 16)] = jnp.zeros((16,), jnp.float32)
    return carry

  lax.fori_loop(0, _HS // 16, zero_zbuf, 0)
  for i in range(8):
    onesb[pl.ds(i * 16, 16)] = jnp.ones((16,), jnp.float32)
  pltpu.sync_copy(zbuf, hv.at[pl.ds(s * _HS, _HS)])
  pltpu.sync_copy(zbuf, he.at[pl.ds(s * _HS, _HS)])
  plsc.subcore_barrier()

  g0, g1 = _grp_range(wid)

  def body(g, carry):
    pltpu.sync_copy(nidx.at[pl.ds(g * _G, _G), :], nbuf)
    pltpu.sync_copy(eidx.at[pl.ds(g * _G, _G), :], ebuf)
    for j in range(_G):
      pltpu.sync_copy(onesb, hv.at[nbuf.at[j]], add=True)
      pltpu.sync_copy(onesb, he.at[ebuf.at[j]], add=True)
    return carry

  lax.fori_loop(g0, g1, body, 0)
  plsc.subcore_barrier()
  base = c * (2 * _NH)
  pltpu.sync_copy(hv.at[pl.ds(s * _HS, _HS)], out.at[pl.ds(base + s * _HS, _HS)])
  pltpu.sync_copy(
      he.at[pl.ds(s * _HS, _HS)], out.at[pl.ds(base + _NH + s * _HS, _HS)]
  )


_hist_call = pl.kernel(
    _hist_body,
    out_type=jax.ShapeDtypeStruct((NC * 2 * _NH,), jnp.float32),
    mesh=_mesh,
    scratch_types=[
        pltpu.VMEM_SHARED((_NH,), jnp.float32),
        pltpu.VMEM_SHARED((_NH,), jnp.float32),
        pltpu.VMEM((_HS,), jnp.float32),
        pltpu.VMEM((128,), jnp.float32),
        pltpu.VMEM((_G, 128), jnp.int32),
        pltpu.VMEM((_G, 128), jnp.int32),
    ],
)


def _scatter_body(nidx, eidx, norm, zer, out, S, nbuf, ebuf, rows):
  c = lax.axis_index("c")
  s = lax.axis_index("s")
  wid = c * NS + s

  pltpu.sync_copy(zer.at[pl.ds(s * _SR, _SR), :], S.at[pl.ds(s * _SR, _SR), :])
  plsc.subcore_barrier()

  g0, g1 = _grp_range(wid)

  def body(g, carry):
    pltpu.sync_copy(nidx.at[pl.ds(g * _G, _G), :], nbuf)
    pltpu.sync_copy(eidx.at[pl.ds(g * _G, _G), :], ebuf)
    for j in range(_G):
      pltpu.sync_copy(norm.at[nbuf.at[j]], rows)
      pltpu.sync_copy(rows, S.at[ebuf.at[j]], add=True)
    return carry

  lax.fori_loop(g0, g1, body, 0)
  plsc.subcore_barrier()
  pltpu.sync_copy(
      S.at[pl.ds(s * _SR, _SR), :], out.at[c, pl.ds(s * _SR, _SR), :]
  )


_scatter_call = pl.kernel(
    _scatter_body,
    out_type=jax.ShapeDtypeStruct((NC, _NP, _K), jnp.float32),
    mesh=_mesh,
    scratch_types=[
        pltpu.VMEM_SHARED((_NP, _K), jnp.float32),
        pltpu.VMEM((_G, 128), jnp.int32),
        pltpu.VMEM((_G, 128), jnp.int32),
        pltpu.VMEM((128, _K), jnp.float32),
    ],
    compiler_params=pltpu.CompilerParams(use_tc_tiling_on_sc=False),
)

_BN = 1000
_GN = _N // _BN


def _norm_body(dvp, dep, z, norm_o, dei_o, dvc_o):
  dv = dvp[:, 0:1] + dvp[:, 1:2]
  de = dep[:, 0:1] + dep[:, 1:2]
  dvc = jnp.maximum(dv, 1e-06)
  dec = jnp.maximum(de, 1e-06)
  dvis = lax.rsqrt(dvc)
  norm_o[...] = z[...] * dvis
  dei_o[...] = 1.0 / dec
  dvc_o[...] = dvc


_norm_call = pl.pallas_call(
    _norm_body,
    grid=(_GN,),
    in_specs=[
        pl.BlockSpec((_BN, 2), lambda i: (i, 0)),
        pl.BlockSpec((_BN, 2), lambda i: (i, 0)),
        pl.BlockSpec((_BN, _K), lambda i: (i, 0)),
    ],
    out_specs=[
        pl.BlockSpec((_BN, _K), lambda i: (i, 0)),
        pl.BlockSpec((_BN, 1), lambda i: (i, 0)),
        pl.BlockSpec((_BN, 1), lambda i: (i, 0)),
    ],
    out_shape=[
        jax.ShapeDtypeStruct((_N, _K), jnp.float32),
        jax.ShapeDtypeStruct((_N, 1), jnp.float32),
        jax.ShapeDtypeStruct((_N, 1), jnp.float32),
    ],
)


def _final_body(sp, dei, dvc, z, out, tacc, facc, ps, ns, pc, nc):
  i = pl.program_id(0)

  @pl.when(i == 0)
  def _init():
    tacc[...] = jnp.zeros_like(tacc)
    facc[...] = jnp.zeros_like(facc)
    ps[0, 0] = 0.0
    ns[0, 0] = 0.0
    pc[0, 0] = 0.0
    nc[0, 0] = 0.0

  s_sum = sp[0] + sp[1]
  tacc[...] += jnp.sum(s_sum * s_sum * dei[...], axis=0, keepdims=True)
  facc[...] += jnp.sum(z[...] * z[...] * dvc[...], axis=0, keepdims=True)
  f = z[:, 0:1]
  pos = f > 0.0
  neg = f < 0.0
  ps[0, 0] += jnp.sum(jnp.where(pos, f, 0.0))
  ns[0, 0] += jnp.sum(jnp.where(neg, f, 0.0))
  pc[0, 0] += jnp.sum(pos.astype(jnp.float32))
  nc[0, 0] += jnp.sum(neg.astype(jnp.float32))

  @pl.when(i == _GN - 1)
  def _fin():
    theta = tacc[...]
    fdvf = jnp.maximum(facc[...], 1e-06)
    rq = 1.0 - theta / fdvf
    rq = jnp.where(jnp.isnan(rq) | jnp.isinf(rq), 0.0, rq)
    rayleigh = jnp.sum(rq) / _K
    pos_mean = ps[0, 0] / jnp.maximum(pc[0, 0], 1.0)
    neg_mean = ns[0, 0] / jnp.maximum(nc[0, 0], 1.0)
    separation = jnp.abs(pos_mean - neg_mean)
    penalty = jnp.where(
        (pc[0, 0] == 0.0) | (nc[0, 0] == 0.0), 1.0, 1.0 / (separation + 1e-06)
    )
    out[0, 0] = rayleigh + 0.1 * penalty


_final_call = pl.pallas_call(
    _final_body,
    grid=(_GN,),
    in_specs=[
        pl.BlockSpec((NC, _BN, _K), lambda i: (0, i, 0)),
        pl.BlockSpec((_BN, 1), lambda i: (i, 0)),
        pl.BlockSpec((_BN, 1), lambda i: (i, 0)),
        pl.BlockSpec((_BN, _K), lambda i: (i, 0)),
    ],
    out_specs=pl.BlockSpec(memory_space=pltpu.SMEM),
    out_shape=jax.ShapeDtypeStruct((1, 1), jnp.float32),
    scratch_shapes=[
        pltpu.VMEM((1, _K), jnp.float32),
        pltpu.VMEM((1, _K), jnp.float32),
        pltpu.SMEM((1, 1), jnp.float32),
        pltpu.SMEM((1, 1), jnp.float32),
        pltpu.SMEM((1, 1), jnp.float32),
        pltpu.SMEM((1, 1), jnp.float32),
    ],
)


@jax.jit
def _run(Z, hyperedge_index):
  nidx2 = hyperedge_index[0].reshape(_R, 128)
  eidx2 = hyperedge_index[1].reshape(_R, 128)
  hist = _hist_call(nidx2, eidx2).reshape(NC, 2, _NH)
  dvp = hist[:, 0, :_N].T
  dep = hist[:, 1, :_N].T
  norm, dei, dvc = _norm_call(dvp, dep, Z)
  zeros_nk = jnp.zeros((_NP, _K), jnp.float32)
  sp = _scatter_call(nidx2, eidx2, norm, zeros_nk)
  out = _final_call(sp, dei, dvc, Z)
  return out[0, 0]


def kernel(Z, hyperedge_index, num_nodes):
  return _run(Z, hyperedge_index) + 0.0 * jnp.asarray(num_nodes, jnp.float32)


# speedup vs baseline: 29.1541x; 1.5505x over previous
import functools

import jax
import jax.numpy as jnp
from jax import lax
from jax.experimental import pallas as pl
from jax.experimental.pallas import tpu as pltpu
from jax.experimental.pallas import tpu_sc as plsc

NC = 2
NS = 16
NW = NC * NS

_N = 100000
_K = 8
_E = 3200000
_CH = 1024
_NGRP = _E // _CH
_NH = 102400
_HS = _NH // NS
_NP = 100096
_SR = _NP // NS

_mesh = plsc.VectorSubcoreMesh(
    core_axis_name="c", subcore_axis_name="s", num_cores=NC, num_subcores=NS
)


def _grp_range(wid):
  g0 = (_NGRP * wid) // NW
  g1 = (_NGRP * (wid + 1)) // NW
  return g0, g1


def _hist_body(nidx, eidx, out, hv, he, zbuf, onesb, nbuf, ebuf):
  c = lax.axis_index("c")
  s = lax.axis_index("s")
  wid = c * NS + s

  def zero_zbuf(i, carry):
    zbuf[pl.ds(i * 16, 16)] = jnp.zeros((16,), jnp.float32)
    return carry

  lax.fori_loop(0, _HS // 16, zero_zbuf, 0)

  def fill_ones(i, carry):
    onesb[pl.ds(i * 16, 16)] = jnp.ones((16,), jnp.float32)
    return carry

  lax.fori_loop(0, _CH // 16, fill_ones, 0)
  pltpu.sync_copy(zbuf, hv.at[pl.ds(s * _HS, _HS)])
  pltpu.sync_copy(zbuf, he.at[pl.ds(s * _HS, _HS)])
  plsc.subcore_barrier()

  g0, g1 = _grp_range(wid)

  def body(g, carry):
    pltpu.sync_copy(nidx.at[pl.ds(g * _CH, _CH)], nbuf)
    pltpu.sync_copy(eidx.at[pl.ds(g * _CH, _CH)], ebuf)
    pltpu.sync_copy(onesb, hv.at[nbuf], add=True)
    pltpu.sync_copy(onesb, he.at[ebuf], add=True)
    return carry

  lax.fori_loop(g0, g1, body, 0)
  plsc.subcore_barrier()
  base = c * (2 * _NH)
  pltpu.sync_copy(hv.at[pl.ds(s * _HS, _HS)], out.at[pl.ds(base + s * _HS, _HS)])
  pltpu.sync_copy(
      he.at[pl.ds(s * _HS, _HS)], out.at[pl.ds(base + _NH + s * _HS, _HS)]
  )


_hist_call = pl.kernel(
    _hist_body,
    out_type=jax.ShapeDtypeStruct((NC * 2 * _NH,), jnp.float32),
    mesh=_mesh,
    scratch_types=[
        pltpu.VMEM_SHARED((_NH,), jnp.float32),
        pltpu.VMEM_SHARED((_NH,), jnp.float32),
        pltpu.VMEM((_HS,), jnp.float32),
        pltpu.VMEM((_CH,), jnp.float32),
        pltpu.VMEM((_CH,), jnp.int32),
        pltpu.VMEM((_CH,), jnp.int32),
    ],
)


def _scatter_body(nidx, eidx, norm, zer, out, S, nbuf, ebuf, rows):
  c = lax.axis_index("c")
  s = lax.axis_index("s")
  wid = c * NS + s

  pltpu.sync_copy(zer.at[pl.ds(s * _SR, _SR), :], S.at[pl.ds(s * _SR, _SR), :])
  plsc.subcore_barrier()

  g0, g1 = _grp_range(wid)

  def body(g, carry):
    pltpu.sync_copy(nidx.at[pl.ds(g * _CH, _CH)], nbuf)
    pltpu.sync_copy(eidx.at[pl.ds(g * _CH, _CH)], ebuf)
    pltpu.sync_copy(norm.at[nbuf], rows)
    pltpu.sync_copy(rows, S.at[ebuf], add=True)
    return carry

  lax.fori_loop(g0, g1, body, 0)
  plsc.subcore_barrier()
  pltpu.sync_copy(
      S.at[pl.ds(s * _SR, _SR), :], out.at[c, pl.ds(s * _SR, _SR), :]
  )


_scatter_call = pl.kernel(
    _scatter_body,
    out_type=jax.ShapeDtypeStruct((NC, _NP, _K), jnp.float32),
    mesh=_mesh,
    scratch_types=[
        pltpu.VMEM_SHARED((_NP, _K), jnp.float32),
        pltpu.VMEM((_CH,), jnp.int32),
        pltpu.VMEM((_CH,), jnp.int32),
        pltpu.VMEM((_CH, _K), jnp.float32),
    ],
    compiler_params=pltpu.CompilerParams(use_tc_tiling_on_sc=False),
)

_BN = 1000
_GN = _N // _BN


def _norm_body(dvp, dep, z, norm_o, dei_o, dvc_o):
  dv = dvp[:, 0:1] + dvp[:, 1:2]
  de = dep[:, 0:1] + dep[:, 1:2]
  dvc = jnp.maximum(dv, 1e-06)
  dec = jnp.maximum(de, 1e-06)
  dvis = lax.rsqrt(dvc)
  norm_o[...] = z[...] * dvis
  dei_o[...] = 1.0 / dec
  dvc_o[...] = dvc


_norm_call = pl.pallas_call(
    _norm_body,
    grid=(_GN,),
    in_specs=[
        pl.BlockSpec((_BN, 2), lambda i: (i, 0)),
        pl.BlockSpec((_BN, 2), lambda i: (i, 0)),
        pl.BlockSpec((_BN, _K), lambda i: (i, 0)),
    ],
    out_specs=[
        pl.BlockSpec((_BN, _K), lambda i: (i, 0)),
        pl.BlockSpec((_BN, 1), lambda i: (i, 0)),
        pl.BlockSpec((_BN, 1), lambda i: (i, 0)),
    ],
    out_shape=[
        jax.ShapeDtypeStruct((_N, _K), jnp.float32),
        jax.ShapeDtypeStruct((_N, 1), jnp.float32),
        jax.ShapeDtypeStruct((_N, 1), jnp.float32),
    ],
)


def _final_body(sp, dei, dvc, z, out, tacc, facc, ps, ns, pc, nc):
  i = pl.program_id(0)

  @pl.when(i == 0)
  def _init():
    tacc[...] = jnp.zeros_like(tacc)
    facc[...] = jnp.zeros_like(facc)
    ps[0, 0] = 0.0
    ns[0, 0] = 0.0
    pc[0, 0] = 0.0
    nc[0, 0] = 0.0

  s_sum = sp[0] + sp[1]
  tacc[...] += jnp.sum(s_sum * s_sum * dei[...], axis=0, keepdims=True)
  facc[...] += jnp.sum(z[...] * z[...] * dvc[...], axis=0, keepdims=True)
  f = z[:, 0:1]
  pos = f > 0.0
  neg = f < 0.0
  ps[0, 0] += jnp.sum(jnp.where(pos, f, 0.0))
  ns[0, 0] += jnp.sum(jnp.where(neg, f, 0.0))
  pc[0, 0] += jnp.sum(pos.astype(jnp.float32))
  nc[0, 0] += jnp.sum(neg.astype(jnp.float32))

  @pl.when(i == _GN - 1)
  def _fin():
    theta = tacc[...]
    fdvf = jnp.maximum(facc[...], 1e-06)
    rq = 1.0 - theta / fdvf
    rq = jnp.where(jnp.isnan(rq) | jnp.isinf(rq), 0.0, rq)
    rayleigh = jnp.sum(rq) / _K
    pos_mean = ps[0, 0] / jnp.maximum(pc[0, 0], 1.0)
    neg_mean = ns[0, 0] / jnp.maximum(nc[0, 0], 1.0)
    separation = jnp.abs(pos_mean - neg_mean)
    penalty = jnp.where(
        (pc[0, 0] == 0.0) | (nc[0, 0] == 0.0), 1.0, 1.0 / (separation + 1e-06)
    )
    out[0, 0] = rayleigh + 0.1 * penalty


_final_call = pl.pallas_call(
    _final_body,
    grid=(_GN,),
    in_specs=[
        pl.BlockSpec((NC, _BN, _K), lambda i: (0, i, 0)),
        pl.BlockSpec((_BN, 1), lambda i: (i, 0)),
        pl.BlockSpec((_BN, 1), lambda i: (i, 0)),
        pl.BlockSpec((_BN, _K), lambda i: (i, 0)),
    ],
    out_specs=pl.BlockSpec(memory_space=pltpu.SMEM),
    out_shape=jax.ShapeDtypeStruct((1, 1), jnp.float32),
    scratch_shapes=[
        pltpu.VMEM((1, _K), jnp.float32),
        pltpu.VMEM((1, _K), jnp.float32),
        pltpu.SMEM((1, 1), jnp.float32),
        pltpu.SMEM((1, 1), jnp.float32),
        pltpu.SMEM((1, 1), jnp.float32),
        pltpu.SMEM((1, 1), jnp.float32),
    ],
)


@jax.jit
def _run(Z, hyperedge_index):
  nidx2 = hyperedge_index[0]
  eidx2 = hyperedge_index[1]
  hist = _hist_call(nidx2, eidx2).reshape(NC, 2, _NH)
  dvp = hist[:, 0, :_N].T
  dep = hist[:, 1, :_N].T
  norm, dei, dvc = _norm_call(dvp, dep, Z)
  zeros_nk = jnp.zeros((_NP, _K), jnp.float32)
  sp = _scatter_call(nidx2, eidx2, norm, zeros_nk)
  out = _final_call(sp, dei, dvc, Z)
  return out[0, 0]


def kernel(Z, hyperedge_index, num_nodes):
  return _run(Z, hyperedge_index) + 0.0 * jnp.asarray(num_nodes, jnp.float32)


# speedup vs baseline: 35.9237x; 1.2322x over previous
import functools

import jax
import jax.numpy as jnp
from jax import lax
from jax.experimental import pallas as pl
from jax.experimental.pallas import tpu as pltpu
from jax.experimental.pallas import tpu_sc as plsc

NC = 2
NS = 16
NW = NC * NS

_N = 100000
_K = 8
_E = 3200000
_CH = 3200
_NGRP = _E // _CH
_NH = 102400
_HS = _NH // NS
_NP = 100096
_SR = _NP // NS

_mesh = plsc.VectorSubcoreMesh(
    core_axis_name="c", subcore_axis_name="s", num_cores=NC, num_subcores=NS
)


def _grp_range(wid):
  g0 = (_NGRP * wid) // NW
  g1 = (_NGRP * (wid + 1)) // NW
  return g0, g1


def _hist_body(nidx, eidx, out, hv, he, zbuf, onesb, nbuf, ebuf):
  c = lax.axis_index("c")
  s = lax.axis_index("s")
  wid = c * NS + s

  def zero_zbuf(i, carry):
    zbuf[pl.ds(i * 16, 16)] = jnp.zeros((16,), jnp.float32)
    return carry

  lax.fori_loop(0, _HS // 16, zero_zbuf, 0)

  def fill_ones(i, carry):
    onesb[pl.ds(i * 16, 16)] = jnp.ones((16,), jnp.float32)
    return carry

  lax.fori_loop(0, _CH // 16, fill_ones, 0)
  pltpu.sync_copy(zbuf, hv.at[pl.ds(s * _HS, _HS)])
  pltpu.sync_copy(zbuf, he.at[pl.ds(s * _HS, _HS)])
  plsc.subcore_barrier()

  g0, g1 = _grp_range(wid)

  def body(g, carry):
    pltpu.sync_copy(nidx.at[pl.ds(g * _CH, _CH)], nbuf)
    pltpu.sync_copy(eidx.at[pl.ds(g * _CH, _CH)], ebuf)
    pltpu.sync_copy(onesb, hv.at[nbuf], add=True)
    pltpu.sync_copy(onesb, he.at[ebuf], add=True)
    return carry

  lax.fori_loop(g0, g1, body, 0)
  plsc.subcore_barrier()
  base = c * (2 * _NH)
  pltpu.sync_copy(hv.at[pl.ds(s * _HS, _HS)], out.at[pl.ds(base + s * _HS, _HS)])
  pltpu.sync_copy(
      he.at[pl.ds(s * _HS, _HS)], out.at[pl.ds(base + _NH + s * _HS, _HS)]
  )


_hist_call = pl.kernel(
    _hist_body,
    out_type=jax.ShapeDtypeStruct((NC * 2 * _NH,), jnp.float32),
    mesh=_mesh,
    scratch_types=[
        pltpu.VMEM_SHARED((_NH,), jnp.float32),
        pltpu.VMEM_SHARED((_NH,), jnp.float32),
        pltpu.VMEM((_HS,), jnp.float32),
        pltpu.VMEM((_CH,), jnp.float32),
        pltpu.VMEM((_CH,), jnp.int32),
        pltpu.VMEM((_CH,), jnp.int32),
    ],
)


def _scatter_body(nidx, eidx, norm, zer, out, S, nbuf, ebuf, rows):
  c = lax.axis_index("c")
  s = lax.axis_index("s")
  wid = c * NS + s

  pltpu.sync_copy(zer.at[pl.ds(s * _SR, _SR), :], S.at[pl.ds(s * _SR, _SR), :])
  plsc.subcore_barrier()

  g0, g1 = _grp_range(wid)

  def body(g, carry):
    pltpu.sync_copy(nidx.at[pl.ds(g * _CH, _CH)], nbuf)
    pltpu.sync_copy(eidx.at[pl.ds(g * _CH, _CH)], ebuf)
    pltpu.sync_copy(norm.at[nbuf], rows)
    pltpu.sync_copy(rows, S.at[ebuf], add=True)
    return carry

  lax.fori_loop(g0, g1, body, 0)
  plsc.subcore_barrier()
  pltpu.sync_copy(
      S.at[pl.ds(s * _SR, _SR), :], out.at[c, pl.ds(s * _SR, _SR), :]
  )


_scatter_call = pl.kernel(
    _scatter_body,
    out_type=jax.ShapeDtypeStruct((NC, _NP, _K), jnp.float32),
    mesh=_mesh,
    scratch_types=[
        pltpu.VMEM_SHARED((_NP, _K), jnp.float32),
        pltpu.VMEM((_CH,), jnp.int32),
        pltpu.VMEM((_CH,), jnp.int32),
        pltpu.VMEM((_CH, _K), jnp.float32),
    ],
    compiler_params=pltpu.CompilerParams(use_tc_tiling_on_sc=False),
)

_BN = 1000
_GN = _N // _BN


def _norm_body(dvp, dep, z, norm_o, dei_o, dvc_o):
  dv = dvp[:, 0:1] + dvp[:, 1:2]
  de = dep[:, 0:1] + dep[:, 1:2]
  dvc = jnp.maximum(dv, 1e-06)
  dec = jnp.maximum(de, 1e-06)
  dvis = lax.rsqrt(dvc)
  norm_o[...] = z[...] * dvis
  dei_o[...] = 1.0 / dec
  dvc_o[...] = dvc


_norm_call = pl.pallas_call(
    _norm_body,
    grid=(_GN,),
    in_specs=[
        pl.BlockSpec((_BN, 2), lambda i: (i, 0)),
        pl.BlockSpec((_BN, 2), lambda i: (i, 0)),
        pl.BlockSpec((_BN, _K), lambda i: (i, 0)),
    ],
    out_specs=[
        pl.BlockSpec((_BN, _K), lambda i: (i, 0)),
        pl.BlockSpec((_BN, 1), lambda i: (i, 0)),
        pl.BlockSpec((_BN, 1), lambda i: (i, 0)),
    ],
    out_shape=[
        jax.ShapeDtypeStruct((_N, _K), jnp.float32),
        jax.ShapeDtypeStruct((_N, 1), jnp.float32),
        jax.ShapeDtypeStruct((_N, 1), jnp.float32),
    ],
)


def _final_body(sp, dei, dvc, z, out, tacc, facc, ps, ns, pc, nc):
  i = pl.program_id(0)

  @pl.when(i == 0)
  def _init():
    tacc[...] = jnp.zeros_like(tacc)
    facc[...] = jnp.zeros_like(facc)
    ps[0, 0] = 0.0
    ns[0, 0] = 0.0
    pc[0, 0] = 0.0
    nc[0, 0] = 0.0

  s_sum = sp[0] + sp[1]
  tacc[...] += jnp.sum(s_sum * s_sum * dei[...], axis=0, keepdims=True)
  facc[...] += jnp.sum(z[...] * z[...] * dvc[...], axis=0, keepdims=True)
  f = z[:, 0:1]
  pos = f > 0.0
  neg = f < 0.0
  ps[0, 0] += jnp.sum(jnp.where(pos, f, 0.0))
  ns[0, 0] += jnp.sum(jnp.where(neg, f, 0.0))
  pc[0, 0] += jnp.sum(pos.astype(jnp.float32))
  nc[0, 0] += jnp.sum(neg.astype(jnp.float32))

  @pl.when(i == _GN - 1)
  def _fin():
    theta = tacc[...]
    fdvf = jnp.maximum(facc[...], 1e-06)
    rq = 1.0 - theta / fdvf
    rq = jnp.where(jnp.isnan(rq) | jnp.isinf(rq), 0.0, rq)
    rayleigh = jnp.sum(rq) / _K
    pos_mean = ps[0, 0] / jnp.maximum(pc[0, 0], 1.0)
    neg_mean = ns[0, 0] / jnp.maximum(nc[0, 0], 1.0)
    separation = jnp.abs(pos_mean - neg_mean)
    penalty = jnp.where(
        (pc[0, 0] == 0.0) | (nc[0, 0] == 0.0), 1.0, 1.0 / (separation + 1e-06)
    )
    out[0, 0] = rayleigh + 0.1 * penalty


_final_call = pl.pallas_call(
    _final_body,
    grid=(_GN,),
    in_specs=[
        pl.BlockSpec((NC, _BN, _K), lambda i: (0, i, 0)),
        pl.BlockSpec((_BN, 1), lambda i: (i, 0)),
        pl.BlockSpec((_BN, 1), lambda i: (i, 0)),
        pl.BlockSpec((_BN, _K), lambda i: (i, 0)),
    ],
    out_specs=pl.BlockSpec(memory_space=pltpu.SMEM),
    out_shape=jax.ShapeDtypeStruct((1, 1), jnp.float32),
    scratch_shapes=[
        pltpu.VMEM((1, _K), jnp.float32),
        pltpu.VMEM((1, _K), jnp.float32),
        pltpu.SMEM((1, 1), jnp.float32),
        pltpu.SMEM((1, 1), jnp.float32),
        pltpu.SMEM((1, 1), jnp.float32),
        pltpu.SMEM((1, 1), jnp.float32),
    ],
)


@jax.jit
def _run(Z, hyperedge_index):
  nidx2 = hyperedge_index[0]
  eidx2 = hyperedge_index[1]
  hist = _hist_call(nidx2, eidx2).reshape(NC, 2, _NH)
  dvp = hist[:, 0, :_N].T
  dep = hist[:, 1, :_N].T
  norm, dei, dvc = _norm_call(dvp, dep, Z)
  zeros_nk = jnp.zeros((_NP, _K), jnp.float32)
  sp = _scatter_call(nidx2, eidx2, norm, zeros_nk)
  out = _final_call(sp, dei, dvc, Z)
  return out[0, 0]


def kernel(Z, hyperedge_index, num_nodes):
  return _run(Z, hyperedge_index) + 0.0 * jnp.asarray(num_nodes, jnp.float32)


# speedup vs baseline: 66.0537x; 1.8387x over previous
import jax
import jax.numpy as jnp
from jax import lax
from jax.experimental import pallas as pl
from jax.experimental.pallas import tpu as pltpu
from jax.experimental.pallas import tpu_sc as plsc

NC = 2
NS = 16
NW = NC * NS

_N = 100000
_K = 8
_E = 3200000
_CH = 3200
_NGRP = _E // _CH
_NH = 102400
_HS = _NH // NS
_NP = 100096
_SR = _NP // NS

_mesh = plsc.VectorSubcoreMesh(
    core_axis_name="c", subcore_axis_name="s", num_cores=NC, num_subcores=NS
)


def _grp_range(wid):
  g0 = (_NGRP * wid) // NW
  g1 = (_NGRP * (wid + 1)) // NW
  return g0, g1


def _hist_body(nidx, eidx, out, hv, he, zbuf, onesb, nbuf, ebuf):
  c = lax.axis_index("c")
  s = lax.axis_index("s")
  wid = c * NS + s

  def zero_zbuf(i, carry):
    zbuf[pl.ds(i * 16, 16)] = jnp.zeros((16,), jnp.float32)
    return carry

  lax.fori_loop(0, _HS // 16, zero_zbuf, 0)

  def fill_ones(i, carry):
    onesb[pl.ds(i * 16, 16)] = jnp.ones((16,), jnp.float32)
    return carry

  lax.fori_loop(0, _CH // 16, fill_ones, 0)
  pltpu.sync_copy(zbuf, hv.at[pl.ds(s * _HS, _HS)])
  pltpu.sync_copy(zbuf, he.at[pl.ds(s * _HS, _HS)])
  plsc.subcore_barrier()

  g0, g1 = _grp_range(wid)

  def body(g, carry):
    pltpu.sync_copy(nidx.at[pl.ds(g * _CH, _CH)], nbuf)
    pltpu.sync_copy(eidx.at[pl.ds(g * _CH, _CH)], ebuf)
    pltpu.sync_copy(onesb, hv.at[nbuf], add=True)
    pltpu.sync_copy(onesb, he.at[ebuf], add=True)
    return carry

  lax.fori_loop(g0, g1, body, 0)
  plsc.subcore_barrier()
  base = c * (2 * _NH)
  pltpu.sync_copy(hv.at[pl.ds(s * _HS, _HS)], out.at[pl.ds(base + s * _HS, _HS)])
  pltpu.sync_copy(
      he.at[pl.ds(s * _HS, _HS)], out.at[pl.ds(base + _NH + s * _HS, _HS)]
  )


_hist_call = pl.kernel(
    _hist_body,
    out_type=jax.ShapeDtypeStruct((NC * 2 * _NH,), jnp.float32),
    mesh=_mesh,
    scratch_types=[
        pltpu.VMEM_SHARED((_NH,), jnp.float32),
        pltpu.VMEM_SHARED((_NH,), jnp.float32),
        pltpu.VMEM((_HS,), jnp.float32),
        pltpu.VMEM((_CH,), jnp.float32),
        pltpu.VMEM((_CH,), jnp.int32),
        pltpu.VMEM((_CH,), jnp.int32),
    ],
)


def _scatter_body(nidx, eidx, norm, zer, out, S, nbuf, ebuf, rows):
  c = lax.axis_index("c")
  s = lax.axis_index("s")
  wid = c * NS + s

  pltpu.sync_copy(zer, S.at[pl.ds(s * _SR, _SR), :])
  plsc.subcore_barrier()

  g0, g1 = _grp_range(wid)

  def body(g, carry):
    pltpu.sync_copy(nidx.at[pl.ds(g * _CH, _CH)], nbuf)
    pltpu.sync_copy(eidx.at[pl.ds(g * _CH, _CH)], ebuf)
    pltpu.sync_copy(norm.at[nbuf], rows)
    pltpu.sync_copy(rows, S.at[ebuf], add=True)
    return carry

  lax.fori_loop(g0, g1, body, 0)
  plsc.subcore_barrier()
  pltpu.sync_copy(
      S.at[pl.ds(s * _SR, _SR), :], out.at[c, pl.ds(s * _SR, _SR), :]
  )


_scatter_call = pl.kernel(
    _scatter_body,
    out_type=jax.ShapeDtypeStruct((NC, _NP, _K), jnp.float32),
    mesh=_mesh,
    scratch_types=[
        pltpu.VMEM_SHARED((_NP, _K), jnp.float32),
        pltpu.VMEM((_CH,), jnp.int32),
        pltpu.VMEM((_CH,), jnp.int32),
        pltpu.VMEM((_CH, _K), jnp.float32),
    ],
    compiler_params=pltpu.CompilerParams(use_tc_tiling_on_sc=False),
)

_NR = _N // 16
_NR2 = _NP // 16


def _lane_repeat_mat():
  li = lax.broadcasted_iota(jnp.int32, (16, 128), 1)
  ri = lax.broadcasted_iota(jnp.int32, (16, 128), 0)
  return (li // 8 == ri).astype(jnp.float32)


def _lane_fold_mat():
  li = lax.broadcasted_iota(jnp.int32, (128, 8), 0)
  ci = lax.broadcasted_iota(jnp.int32, (128, 8), 1)
  return (li % 8 == ci).astype(jnp.float32)


def _norm_body(dvp0, dvp1, dep0, dep1, z2, norm_o, dvc_o, dei_o):
  dv = dvp0[: _NR] + dvp1[: _NR]
  de = dep0[: _NR] + dep1[: _NR]
  dvc = jnp.maximum(dv, 1e-06)
  dec = jnp.maximum(de, 1e-06)
  dvc_o[...] = dvc
  dei_o[...] = 1.0 / dec
  dvis = lax.rsqrt(dvc)
  rep = jnp.dot(dvis, _lane_repeat_mat(), preferred_element_type=jnp.float32)
  norm_o[...] = z2[...] * rep


_norm_call = pl.pallas_call(
    _norm_body,
    out_shape=[
        jax.ShapeDtypeStruct((_NR, 128), jnp.float32),
        jax.ShapeDtypeStruct((_NR, 16), jnp.float32),
        jax.ShapeDtypeStruct((_NR, 16), jnp.float32),
    ],
)


def _final_body(sp2, z2, dvc, dei, out):
  s2 = sp2[0, : _NR] + sp2[1, : _NR]
  rep = _lane_repeat_mat()
  dei_rep = jnp.dot(dei[...], rep, preferred_element_type=jnp.float32)
  dvc_rep = jnp.dot(dvc[...], rep, preferred_element_type=jnp.float32)
  z = z2[...]
  t128 = jnp.sum(s2 * s2 * dei_rep, axis=0, keepdims=True)
  f128 = jnp.sum(z * z * dvc_rep, axis=0, keepdims=True)
  fold = _lane_fold_mat()
  theta = jnp.dot(t128, fold, preferred_element_type=jnp.float32)
  fdvf = jnp.maximum(
      jnp.dot(f128, fold, preferred_element_type=jnp.float32), 1e-06
  )
  rq = 1.0 - theta / fdvf
  rq = jnp.where(jnp.isnan(rq) | jnp.isinf(rq), 0.0, rq)
  rayleigh = jnp.sum(rq) / _K
  colmask = lax.broadcasted_iota(jnp.int32, (_NR, 128), 1) % 8 == 0
  posm = colmask & (z > 0.0)
  negm = colmask & (z < 0.0)
  ps = jnp.sum(jnp.where(posm, z, 0.0))
  ns = jnp.sum(jnp.where(negm, z, 0.0))
  pc = jnp.sum(posm.astype(jnp.float32))
  nc = jnp.sum(negm.astype(jnp.float32))
  pos_mean = ps / jnp.maximum(pc, 1.0)
  neg_mean = ns / jnp.maximum(nc, 1.0)
  separation = jnp.abs(pos_mean - neg_mean)
  penalty = jnp.where((pc == 0.0) | (nc == 0.0), 1.0, 1.0 / (separation + 1e-06))
  out[0, 0] = rayleigh + 0.1 * penalty


_final_call = pl.pallas_call(
    _final_body,
    out_specs=pl.BlockSpec(memory_space=pltpu.SMEM),
    out_shape=jax.ShapeDtypeStruct((1, 1), jnp.float32),
)


@jax.jit
def _run(Z, hyperedge_index):
  nidx2 = hyperedge_index[0]
  eidx2 = hyperedge_index[1]
  hist = _hist_call(nidx2, eidx2).reshape(NC, 2, _NH)
  dvp0 = hist[0, 0].reshape(_NH // 16, 16)
  dvp1 = hist[1, 0].reshape(_NH // 16, 16)
  dep0 = hist[0, 1].reshape(_NH // 16, 16)
  dep1 = hist[1, 1].reshape(_NH // 16, 16)
  z2 = Z.reshape(_NR, 128)
  norm2, dvc, dei = _norm_call(dvp0, dvp1, dep0, dep1, z2)
  norm = norm2.reshape(_N, _K)
  zeros_nk = jnp.zeros((_SR, _K), jnp.float32)
  sp = _scatter_call(nidx2, eidx2, norm, zeros_nk)
  sp2 = sp.reshape(NC, _NR2, 128)
  out = _final_call(sp2, z2, dvc, dei)
  return out[0, 0]


def kernel(Z, hyperedge_index, num_nodes):
  return _run(Z, hyperedge_index) + 0.0 * jnp.asarray(num_nodes, jnp.float32)


# speedup vs baseline: 93.0403x; 1.4086x over previous
import jax
import jax.numpy as jnp
from jax import lax
from jax.experimental import pallas as pl
from jax.experimental.pallas import tpu as pltpu
from jax.experimental.pallas import tpu_sc as plsc

NC = 2
NS = 16
NW = NC * NS

_N = 100000
_K = 8
_E = 3200000
_CH = 3200
_NGRP = _E // _CH
_NH = 102400
_HS = _NH // NS
_NP = 100096
_SR = _NP // NS

_mesh = plsc.VectorSubcoreMesh(
    core_axis_name="c", subcore_axis_name="s", num_cores=NC, num_subcores=NS
)


def _grp_range(wid):
  g0 = (_NGRP * wid) // NW
  g1 = (_NGRP * (wid + 1)) // NW
  return g0, g1


def _hist_body(nidx, eidx, out, hv, he, zbuf, onesb,
               nb0, nb1, nb2, nb3, eb0, eb1, eb2, eb3,
               sn0, sn1, sn2, sn3, se0, se1, se2, se3,
               shv0, shv1, she0, she1):
  c = lax.axis_index("c")
  s = lax.axis_index("s")
  wid = c * NS + s
  nbufs = [nb0, nb1, nb2, nb3]
  ebufs = [eb0, eb1, eb2, eb3]
  sns = [sn0, sn1, sn2, sn3]
  ses = [se0, se1, se2, se3]
  shv = [shv0, shv1]
  she = [she0, she1]

  def zero_zbuf(i, carry):
    zbuf[pl.ds(i * 16, 16)] = jnp.zeros((16,), jnp.float32)
    return carry

  lax.fori_loop(0, _HS // 16, zero_zbuf, 0)

  def fill_ones(i, carry):
    onesb[pl.ds(i * 16, 16)] = jnp.ones((16,), jnp.float32)
    return carry

  lax.fori_loop(0, _CH // 16, fill_ones, 0)
  pltpu.sync_copy(zbuf, hv.at[pl.ds(s * _HS, _HS)])
  pltpu.sync_copy(zbuf, he.at[pl.ds(s * _HS, _HS)])
  plsc.subcore_barrier()

  q0, q1 = _grp_range(wid)
  n = q1 - q0

  def start_idx(i, u):
    pltpu.async_copy(nidx.at[pl.ds(i * _CH, _CH)], nbufs[u], sns[u])
    pltpu.async_copy(eidx.at[pl.ds(i * _CH, _CH)], ebufs[u], ses[u])

  for u in range(2):
    @pl.when(q0 + u < q1)
    def _():
      start_idx(q0 + u, u)

  def outer(m, carry):
    for u in range(4):
      i = q0 + m * 4 + u

      @pl.when(i < q1)
      def _():
        p = u % 2

        @pl.when(i - 2 >= q0)
        def _():
          pltpu.make_async_copy(onesb, hv.at[nbufs[(u + 2) % 4]], shv[p]).wait()
          pltpu.make_async_copy(onesb, he.at[ebufs[(u + 2) % 4]], she[p]).wait()

        @pl.when(i + 2 < q1)
        def _():
          start_idx(i + 2, (u + 2) % 4)

        pltpu.make_async_copy(nidx.at[pl.ds(i * _CH, _CH)], nbufs[u], sns[u]).wait()
        pltpu.make_async_copy(eidx.at[pl.ds(i * _CH, _CH)], ebufs[u], ses[u]).wait()
        pltpu.async_copy(onesb, hv.at[nbufs[u]], shv[p], add=True)
        pltpu.async_copy(onesb, he.at[ebufs[u]], she[p], add=True)

    return carry

  lax.fori_loop(0, (n + 3) // 4, outer, 0)
  for p in range(2):
    @pl.when(n > p)
    def _():
      pltpu.make_async_copy(onesb, hv.at[nbufs[p]], shv[p]).wait()
      pltpu.make_async_copy(onesb, he.at[ebufs[p]], she[p]).wait()

  plsc.subcore_barrier()
  base = c * (2 * _NH)
  pltpu.sync_copy(hv.at[pl.ds(s * _HS, _HS)], out.at[pl.ds(base + s * _HS, _HS)])
  pltpu.sync_copy(
      he.at[pl.ds(s * _HS, _HS)], out.at[pl.ds(base + _NH + s * _HS, _HS)]
  )


_hist_call = pl.kernel(
    _hist_body,
    out_type=jax.ShapeDtypeStruct((NC * 2 * _NH,), jnp.float32),
    mesh=_mesh,
    scratch_types=[
        pltpu.VMEM_SHARED((_NH,), jnp.float32),
        pltpu.VMEM_SHARED((_NH,), jnp.float32),
        pltpu.VMEM((_HS,), jnp.float32),
        pltpu.VMEM((_CH,), jnp.float32),
    ]
    + [pltpu.VMEM((_CH,), jnp.int32)] * 8
    + [pltpu.SemaphoreType.DMA] * 12,
)


def _scatter_body(nidx, eidx, norm, zer, out, S,
                  nb0, nb1, nb2, nb3, eb0, eb1, eb2, eb3, rows0, rows1,
                  sn0, sn1, sn2, sn3, se0, se1, se2, se3,
                  sg0, sg1, ss0, ss1):
  c = lax.axis_index("c")
  s = lax.axis_index("s")
  wid = c * NS + s
  nbufs = [nb0, nb1, nb2, nb3]
  ebufs = [eb0, eb1, eb2, eb3]
  rows = [rows0, rows1]
  sns = [sn0, sn1, sn2, sn3]
  ses = [se0, se1, se2, se3]
  sg = [sg0, sg1]
  ss = [ss0, ss1]

  pltpu.sync_copy(zer, S.at[pl.ds(s * _SR, _SR), :])
  plsc.subcore_barrier()

  q0, q1 = _grp_range(wid)
  n = q1 - q0

  def start_idx(i, u):
    pltpu.async_copy(nidx.at[pl.ds(i * _CH, _CH)], nbufs[u], sns[u])
    pltpu.async_copy(eidx.at[pl.ds(i * _CH, _CH)], ebufs[u], ses[u])

  for u in range(2):
    @pl.when(q0 + u < q1)
    def _():
      start_idx(q0 + u, u)

  def outer(m, carry):
    for u in range(4):
      i = q0 + m * 4 + u

      @pl.when(i < q1)
      def _():
        p = u % 2

        @pl.when(i - 2 >= q0)
        def _():
          pltpu.make_async_copy(rows[p], S.at[ebufs[(u + 2) % 4]], ss[p]).wait()

        @pl.when(i + 2 < q1)
        def _():
          start_idx(i + 2, (u + 2) % 4)

        pltpu.make_async_copy(nidx.at[pl.ds(i * _CH, _CH)], nbufs[u], sns[u]).wait()
        pltpu.async_copy(norm.at[nbufs[u]], rows[p], sg[p])
        pltpu.make_async_copy(eidx.at[pl.ds(i * _CH, _CH)], ebufs[u], ses[u]).wait()
        pltpu.make_async_copy(norm.at[nbufs[u]], rows[p], sg[p]).wait()
        pltpu.async_copy(rows[p], S.at[ebufs[u]], ss[p], add=True)

    return carry

  lax.fori_loop(0, (n + 3) // 4, outer, 0)
  for p in range(2):
    @pl.when(n > p)
    def _():
      pltpu.make_async_copy(rows[p], S.at[ebufs[p]], ss[p]).wait()

  plsc.subcore_barrier()
  pltpu.sync_copy(
      S.at[pl.ds(s * _SR, _SR), :], out.at[c, pl.ds(s * _SR, _SR), :]
  )


_scatter_call = pl.kernel(
    _scatter_body,
    out_type=jax.ShapeDtypeStruct((NC, _NP, _K), jnp.float32),
    mesh=_mesh,
    scratch_types=[
        pltpu.VMEM_SHARED((_NP, _K), jnp.float32),
    ]
    + [pltpu.VMEM((_CH,), jnp.int32)] * 8
    + [pltpu.VMEM((_CH, _K), jnp.float32)] * 2
    + [pltpu.SemaphoreType.DMA] * 12,
    compiler_params=pltpu.CompilerParams(use_tc_tiling_on_sc=False),
)

_NR = _N // 16
_NR2 = _NP // 16


def _lane_repeat_mat():
  li = lax.broadcasted_iota(jnp.int32, (16, 128), 1)
  ri = lax.broadcasted_iota(jnp.int32, (16, 128), 0)
  return (li // 8 == ri).astype(jnp.float32)


def _lane_fold_mat():
  li = lax.broadcasted_iota(jnp.int32, (128, 8), 0)
  ci = lax.broadcasted_iota(jnp.int32, (128, 8), 1)
  return (li % 8 == ci).astype(jnp.float32)


def _norm_body(dvp0, dvp1, dep0, dep1, z2, norm_o, dvc_o, dei_o):
  dv = dvp0[: _NR] + dvp1[: _NR]
  de = dep0[: _NR] + dep1[: _NR]
  dvc = jnp.maximum(dv, 1e-06)
  dec = jnp.maximum(de, 1e-06)
  dvc_o[...] = dvc
  dei_o[...] = 1.0 / dec
  dvis = lax.rsqrt(dvc)
  rep = jnp.dot(dvis, _lane_repeat_mat(), preferred_element_type=jnp.float32)
  norm_o[...] = z2[...] * rep


_norm_call = pl.pallas_call(
    _norm_body,
    out_shape=[
        jax.ShapeDtypeStruct((_NR, 128), jnp.float32),
        jax.ShapeDtypeStruct((_NR, 16), jnp.float32),
        jax.ShapeDtypeStruct((_NR, 16), jnp.float32),
    ],
)


def _final_body(sp2, z2, dvc, dei, out):
  s2 = sp2[0, : _NR] + sp2[1, : _NR]
  rep = _lane_repeat_mat()
  dei_rep = jnp.dot(dei[...], rep, preferred_element_type=jnp.float32)
  dvc_rep = jnp.dot(dvc[...], rep, preferred_element_type=jnp.float32)
  z = z2[...]
  t128 = jnp.sum(s2 * s2 * dei_rep, axis=0, keepdims=True)
  f128 = jnp.sum(z * z * dvc_rep, axis=0, keepdims=True)
  fold = _lane_fold_mat()
  theta = jnp.dot(t128, fold, preferred_element_type=jnp.float32)
  fdvf = jnp.maximum(
      jnp.dot(f128, fold, preferred_element_type=jnp.float32), 1e-06
  )
  rq = 1.0 - theta / fdvf
  rq = jnp.where(jnp.isnan(rq) | jnp.isinf(rq), 0.0, rq)
  rayleigh = jnp.sum(rq) / _K
  colmask = lax.broadcasted_iota(jnp.int32, (_NR, 128), 1) % 8 == 0
  posm = colmask & (z > 0.0)
  negm = colmask & (z < 0.0)
  ps = jnp.sum(jnp.where(posm, z, 0.0))
  ns = jnp.sum(jnp.where(negm, z, 0.0))
  pc = jnp.sum(posm.astype(jnp.float32))
  nc = jnp.sum(negm.astype(jnp.float32))
  pos_mean = ps / jnp.maximum(pc, 1.0)
  neg_mean = ns / jnp.maximum(nc, 1.0)
  separation = jnp.abs(pos_mean - neg_mean)
  penalty = jnp.where((pc == 0.0) | (nc == 0.0), 1.0, 1.0 / (separation + 1e-06))
  out[0, 0] = rayleigh + 0.1 * penalty


_final_call = pl.pallas_call(
    _final_body,
    out_specs=pl.BlockSpec(memory_space=pltpu.SMEM),
    out_shape=jax.ShapeDtypeStruct((1, 1), jnp.float32),
)


@jax.jit
def _run(Z, hyperedge_index):
  nidx2 = hyperedge_index[0]
  eidx2 = hyperedge_index[1]
  hist = _hist_call(nidx2, eidx2).reshape(NC, 2, _NH)
  dvp0 = hist[0, 0].reshape(_NH // 16, 16)
  dvp1 = hist[1, 0].reshape(_NH // 16, 16)
  dep0 = hist[0, 1].reshape(_NH // 16, 16)
  dep1 = hist[1, 1].reshape(_NH // 16, 16)
  z2 = Z.reshape(_NR, 128)
  norm2, dvc, dei = _norm_call(dvp0, dvp1, dep0, dep1, z2)
  norm = norm2.reshape(_N, _K)
  zeros_nk = jnp.zeros((_SR, _K), jnp.float32)
  sp = _scatter_call(nidx2, eidx2, norm, zeros_nk)
  sp2 = sp.reshape(NC, _NR2, 128)
  out = _final_call(sp2, z2, dvc, dei)
  return out[0, 0]


def kernel(Z, hyperedge_index, num_nodes):
  return _run(Z, hyperedge_index) + 0.0 * jnp.asarray(num_nodes, jnp.float32)


# speedup vs baseline: 97.2634x; 1.0454x over previous
import jax
import jax.numpy as jnp
from jax import lax
from jax.experimental import pallas as pl
from jax.experimental.pallas import tpu as pltpu
from jax.experimental.pallas import tpu_sc as plsc

NC = 2
NS = 16
NW = NC * NS

_N = 100000
_K = 8
_E = 3200000
_CH = 3200
_NGRP = _E // _CH
_NH = 102400
_HS = _NH // NS
_NP = 100096
_SR = _NP // NS

_mesh = plsc.VectorSubcoreMesh(
    core_axis_name="c", subcore_axis_name="s", num_cores=NC, num_subcores=NS
)


def _grp_range(wid):
  g0 = (_NGRP * wid) // NW
  g1 = (_NGRP * (wid + 1)) // NW
  return g0, g1


def _hist_body(hei, out, hv, he, zbuf, onesb,
               nb0, nb1, nb2, nb3, eb0, eb1, eb2, eb3,
               sn0, sn1, sn2, sn3, se0, se1, se2, se3,
               shv0, shv1, she0, she1):
  c = lax.axis_index("c")
  s = lax.axis_index("s")
  wid = c * NS + s
  nbufs = [nb0, nb1, nb2, nb3]
  ebufs = [eb0, eb1, eb2, eb3]
  sns = [sn0, sn1, sn2, sn3]
  ses = [se0, se1, se2, se3]
  shv = [shv0, shv1]
  she = [she0, she1]

  def zero_zbuf(i, carry):
    zbuf[pl.ds(i * 16, 16)] = jnp.zeros((16,), jnp.float32)
    return carry

  lax.fori_loop(0, _HS // 16, zero_zbuf, 0)

  def fill_ones(i, carry):
    onesb[pl.ds(i * 16, 16)] = jnp.ones((16,), jnp.float32)
    return carry

  lax.fori_loop(0, _CH // 16, fill_ones, 0)
  pltpu.sync_copy(zbuf, hv.at[pl.ds(s * _HS, _HS)])
  pltpu.sync_copy(zbuf, he.at[pl.ds(s * _HS, _HS)])
  plsc.subcore_barrier()

  q0, q1 = _grp_range(wid)
  n = q1 - q0

  def start_idx(i, u):
    pltpu.async_copy(hei.at[0, pl.ds(i * _CH, _CH)], nbufs[u], sns[u])
    pltpu.async_copy(hei.at[1, pl.ds(i * _CH, _CH)], ebufs[u], ses[u])

  for u in range(2):
    @pl.when(q0 + u < q1)
    def _():
      start_idx(q0 + u, u)

  def outer(m, carry):
    for u in range(4):
      i = q0 + m * 4 + u

      @pl.when(i < q1)
      def _():
        p = u % 2

        @pl.when(i - 2 >= q0)
        def _():
          pltpu.make_async_copy(onesb, hv.at[nbufs[(u + 2) % 4]], shv[p]).wait()
          pltpu.make_async_copy(onesb, he.at[ebufs[(u + 2) % 4]], she[p]).wait()

        @pl.when(i + 2 < q1)
        def _():
          start_idx(i + 2, (u + 2) % 4)

        pltpu.make_async_copy(hei.at[0, pl.ds(i * _CH, _CH)], nbufs[u], sns[u]).wait()
        pltpu.make_async_copy(hei.at[1, pl.ds(i * _CH, _CH)], ebufs[u], ses[u]).wait()
        pltpu.async_copy(onesb, hv.at[nbufs[u]], shv[p], add=True)
        pltpu.async_copy(onesb, he.at[ebufs[u]], she[p], add=True)

    return carry

  lax.fori_loop(0, (n + 3) // 4, outer, 0)
  for p in range(2):
    @pl.when(n > p)
    def _():
      pltpu.make_async_copy(onesb, hv.at[nbufs[p]], shv[p]).wait()
      pltpu.make_async_copy(onesb, he.at[ebufs[p]], she[p]).wait()

  plsc.subcore_barrier()
  base = c * (2 * _NH)
  pltpu.sync_copy(hv.at[pl.ds(s * _HS, _HS)], out.at[pl.ds(base + s * _HS, _HS)])
  pltpu.sync_copy(
      he.at[pl.ds(s * _HS, _HS)], out.at[pl.ds(base + _NH + s * _HS, _HS)]
  )


_hist_call = pl.kernel(
    _hist_body,
    out_type=jax.ShapeDtypeStruct((NC * 2 * _NH,), jnp.float32),
    mesh=_mesh,
    scratch_types=[
        pltpu.VMEM_SHARED((_NH,), jnp.float32),
        pltpu.VMEM_SHARED((_NH,), jnp.float32),
        pltpu.VMEM((_HS,), jnp.float32),
        pltpu.VMEM((_CH,), jnp.float32),
    ]
    + [pltpu.VMEM((_CH,), jnp.int32)] * 8
    + [pltpu.SemaphoreType.DMA] * 12,
    compiler_params=pltpu.CompilerParams(use_tc_tiling_on_sc=False),
)


def _scatter_body(hei, norm, zer, out, S,
                  nb0, nb1, nb2, nb3, eb0, eb1, eb2, eb3, rows0, rows1,
                  sn0, sn1, sn2, sn3, se0, se1, se2, se3,
                  sg0, sg1, ss0, ss1):
  c = lax.axis_index("c")
  s = lax.axis_index("s")
  wid = c * NS + s
  nbufs = [nb0, nb1, nb2, nb3]
  ebufs = [eb0, eb1, eb2, eb3]
  rows = [rows0, rows1]
  sns = [sn0, sn1, sn2, sn3]
  ses = [se0, se1, se2, se3]
  sg = [sg0, sg1]
  ss = [ss0, ss1]

  pltpu.sync_copy(zer, S.at[pl.ds(s * _SR, _SR), :])
  plsc.subcore_barrier()

  q0, q1 = _grp_range(wid)
  n = q1 - q0

  def start_idx(i, u):
    pltpu.async_copy(hei.at[0, pl.ds(i * _CH, _CH)], nbufs[u], sns[u])
    pltpu.async_copy(hei.at[1, pl.ds(i * _CH, _CH)], ebufs[u], ses[u])

  for u in range(2):
    @pl.when(q0 + u < q1)
    def _():
      start_idx(q0 + u, u)

  def outer(m, carry):
    for u in range(4):
      i = q0 + m * 4 + u

      @pl.when(i < q1)
      def _():
        p = u % 2

        @pl.when(i - 2 >= q0)
        def _():
          pltpu.make_async_copy(rows[p], S.at[ebufs[(u + 2) % 4]], ss[p]).wait()

        @pl.when(i + 2 < q1)
        def _():
          start_idx(i + 2, (u + 2) % 4)

        pltpu.make_async_copy(hei.at[0, pl.ds(i * _CH, _CH)], nbufs[u], sns[u]).wait()
        pltpu.async_copy(norm.at[nbufs[u]], rows[p], sg[p])
        pltpu.make_async_copy(hei.at[1, pl.ds(i * _CH, _CH)], ebufs[u], ses[u]).wait()
        pltpu.make_async_copy(norm.at[nbufs[u]], rows[p], sg[p]).wait()
        pltpu.async_copy(rows[p], S.at[ebufs[u]], ss[p], add=True)

    return carry

  lax.fori_loop(0, (n + 3) // 4, outer, 0)
  for p in range(2):
    @pl.when(n > p)
    def _():
      pltpu.make_async_copy(rows[p], S.at[ebufs[p]], ss[p]).wait()

  plsc.subcore_barrier()
  pltpu.sync_copy(
      S.at[pl.ds(s * _SR, _SR), :], out.at[c, pl.ds(s * _SR, _SR), :]
  )


_scatter_call = pl.kernel(
    _scatter_body,
    out_type=jax.ShapeDtypeStruct((NC, _NP, _K), jnp.float32),
    mesh=_mesh,
    scratch_types=[
        pltpu.VMEM_SHARED((_NP, _K), jnp.float32),
    ]
    + [pltpu.VMEM((_CH,), jnp.int32)] * 8
    + [pltpu.VMEM((_CH, _K), jnp.float32)] * 2
    + [pltpu.SemaphoreType.DMA] * 12,
    compiler_params=pltpu.CompilerParams(use_tc_tiling_on_sc=False),
)

_NR = _N // 16
_NR2 = _NP // 16


def _lane_repeat_mat():
  li = lax.broadcasted_iota(jnp.int32, (16, 128), 1)
  ri = lax.broadcasted_iota(jnp.int32, (16, 128), 0)
  return (li // 8 == ri).astype(jnp.float32)


def _lane_fold_mat():
  li = lax.broadcasted_iota(jnp.int32, (128, 8), 0)
  ci = lax.broadcasted_iota(jnp.int32, (128, 8), 1)
  return (li % 8 == ci).astype(jnp.float32)


_HR = _NH // 16


def _norm_body(hist2, z2, norm_o, dvc_o, dei_o):
  dv = hist2[0 * _HR : 0 * _HR + _NR] + hist2[2 * _HR : 2 * _HR + _NR]
  de = hist2[1 * _HR : 1 * _HR + _NR] + hist2[3 * _HR : 3 * _HR + _NR]
  dvc = jnp.maximum(dv, 1e-06)
  dec = jnp.maximum(de, 1e-06)
  dvc_o[...] = dvc
  dei_o[...] = 1.0 / dec
  dvis = lax.rsqrt(dvc)
  rep = jnp.dot(dvis, _lane_repeat_mat(), preferred_element_type=jnp.float32)
  norm_o[...] = z2[...] * rep


_norm_call = pl.pallas_call(
    _norm_body,
    out_shape=[
        jax.ShapeDtypeStruct((_NR, 128), jnp.float32),
        jax.ShapeDtypeStruct((_NR, 16), jnp.float32),
        jax.ShapeDtypeStruct((_NR, 16), jnp.float32),
    ],
)


def _final_body(sp2, z2, dvc, dei, out):
  s2 = sp2[0, : _NR] + sp2[1, : _NR]
  rep = _lane_repeat_mat()
  dei_rep = jnp.dot(dei[...], rep, preferred_element_type=jnp.float32)
  dvc_rep = jnp.dot(dvc[...], rep, preferred_element_type=jnp.float32)
  z = z2[...]
  t128 = jnp.sum(s2 * s2 * dei_rep, axis=0, keepdims=True)
  f128 = jnp.sum(z * z * dvc_rep, axis=0, keepdims=True)
  fold = _lane_fold_mat()
  theta = jnp.dot(t128, fold, preferred_element_type=jnp.float32)
  fdvf = jnp.maximum(
      jnp.dot(f128, fold, preferred_element_type=jnp.float32), 1e-06
  )
  rq = 1.0 - theta / fdvf
  rq = jnp.where(jnp.isnan(rq) | jnp.isinf(rq), 0.0, rq)
  rayleigh = jnp.sum(rq) / _K
  colmask = lax.broadcasted_iota(jnp.int32, (_NR, 128), 1) % 8 == 0
  posm = colmask & (z > 0.0)
  negm = colmask & (z < 0.0)
  ps = jnp.sum(jnp.where(posm, z, 0.0))
  ns = jnp.sum(jnp.where(negm, z, 0.0))
  pc = jnp.sum(posm.astype(jnp.float32))
  nc = jnp.sum(negm.astype(jnp.float32))
  pos_mean = ps / jnp.maximum(pc, 1.0)
  neg_mean = ns / jnp.maximum(nc, 1.0)
  separation = jnp.abs(pos_mean - neg_mean)
  penalty = jnp.where((pc == 0.0) | (nc == 0.0), 1.0, 1.0 / (separation + 1e-06))
  out[0, 0] = rayleigh + 0.1 * penalty


_final_call = pl.pallas_call(
    _final_body,
    out_specs=pl.BlockSpec(memory_space=pltpu.SMEM),
    out_shape=jax.ShapeDtypeStruct((1, 1), jnp.float32),
)


@jax.jit
def _run(Z, hyperedge_index):
  hist2 = _hist_call(hyperedge_index).reshape(NC * 2 * _NH // 16, 16)
  z2 = Z.reshape(_NR, 128)
  norm2, dvc, dei = _norm_call(hist2, z2)
  norm = norm2.reshape(_N, _K)
  zeros_nk = jnp.zeros((_SR, _K), jnp.float32)
  sp = _scatter_call(hyperedge_index, norm, zeros_nk)
  sp2 = sp.reshape(NC, _NR2, 128)
  out = _final_call(sp2, z2, dvc, dei)
  return out[0, 0]


def kernel(Z, hyperedge_index, num_nodes):
  return _run(Z, hyperedge_index) + 0.0 * jnp.asarray(num_nodes, jnp.float32)


# speedup vs baseline: 101.9488x; 1.0482x over previous
import jax
import jax.numpy as jnp
from jax import lax
from jax.experimental import pallas as pl
from jax.experimental.pallas import tpu as pltpu
from jax.experimental.pallas import tpu_sc as plsc

NC = 2
NS = 16
NW = NC * NS

_N = 100000
_K = 8
_E = 3200000
_CH = 2000
_Q = _E // _CH // NW
_NGRP = _E // _CH
_NH = 102400
_HS = _NH // NS
_NP = 100096
_SR = _NP // NS
_NSR = _N // NS

_mesh = plsc.VectorSubcoreMesh(
    core_axis_name="c", subcore_axis_name="s", num_cores=NC, num_subcores=NS
)


def _grp_range(wid):
  g0 = (_NGRP * wid) // NW
  g1 = (_NGRP * (wid + 1)) // NW
  return g0, g1


def _hist_body(hei, out, hv, he, zbuf, onesb,
               nb0, nb1, nb2, nb3, eb0, eb1, eb2, eb3,
               sn0, sn1, sn2, sn3, se0, se1, se2, se3,
               shv0, shv1, she0, she1):
  c = lax.axis_index("c")
  s = lax.axis_index("s")
  wid = c * NS + s
  nbufs = [nb0, nb1, nb2, nb3]
  ebufs = [eb0, eb1, eb2, eb3]
  sns = [sn0, sn1, sn2, sn3]
  ses = [se0, se1, se2, se3]
  shv = [shv0, shv1]
  she = [she0, she1]

  def zero_zbuf(i, carry):
    zbuf[pl.ds(i * 16, 16)] = jnp.zeros((16,), jnp.float32)
    return carry

  lax.fori_loop(0, _HS // 16, zero_zbuf, 0)

  def fill_ones(i, carry):
    onesb[pl.ds(i * 16, 16)] = jnp.ones((16,), jnp.float32)
    return carry

  lax.fori_loop(0, _CH // 16, fill_ones, 0)
  pltpu.sync_copy(zbuf, hv.at[pl.ds(s * _HS, _HS)])
  pltpu.sync_copy(zbuf, he.at[pl.ds(s * _HS, _HS)])
  plsc.subcore_barrier()

  q0, q1 = _grp_range(wid)
  n = q1 - q0

  def start_idx(i, u):
    pltpu.async_copy(hei.at[0, pl.ds(i * _CH, _CH)], nbufs[u], sns[u])
    pltpu.async_copy(hei.at[1, pl.ds(i * _CH, _CH)], ebufs[u], ses[u])

  for u in range(2):
    @pl.when(q0 + u < q1)
    def _():
      start_idx(q0 + u, u)

  def outer(m, carry):
    for u in range(4):
      i = q0 + m * 4 + u

      @pl.when(i < q1)
      def _():
        p = u % 2

        @pl.when(i - 2 >= q0)
        def _():
          pltpu.make_async_copy(onesb, hv.at[nbufs[(u + 2) % 4]], shv[p]).wait()
          pltpu.make_async_copy(onesb, he.at[ebufs[(u + 2) % 4]], she[p]).wait()

        @pl.when(i + 2 < q1)
        def _():
          start_idx(i + 2, (u + 2) % 4)

        pltpu.make_async_copy(hei.at[0, pl.ds(i * _CH, _CH)], nbufs[u], sns[u]).wait()
        pltpu.make_async_copy(hei.at[1, pl.ds(i * _CH, _CH)], ebufs[u], ses[u]).wait()
        pltpu.async_copy(onesb, hv.at[nbufs[u]], shv[p], add=True)
        pltpu.async_copy(onesb, he.at[ebufs[u]], she[p], add=True)

    return carry

  lax.fori_loop(0, (n + 3) // 4, outer, 0)
  for p in range(2):
    @pl.when(n > p)
    def _():
      pltpu.make_async_copy(onesb, hv.at[nbufs[p]], shv[p]).wait()
      pltpu.make_async_copy(onesb, he.at[ebufs[p]], she[p]).wait()

  plsc.subcore_barrier()
  base = c * (2 * _NH)
  pltpu.sync_copy(hv.at[pl.ds(s * _HS, _HS)], out.at[pl.ds(base + s * _HS, _HS)])
  pltpu.sync_copy(
      he.at[pl.ds(s * _HS, _HS)], out.at[pl.ds(base + _NH + s * _HS, _HS)]
  )


_hist_call = pl.kernel(
    _hist_body,
    out_type=jax.ShapeDtypeStruct((NC * 2 * _NH,), jnp.float32),
    mesh=_mesh,
    scratch_types=[
        pltpu.VMEM_SHARED((_NH,), jnp.float32),
        pltpu.VMEM_SHARED((_NH,), jnp.float32),
        pltpu.VMEM((_HS,), jnp.float32),
        pltpu.VMEM((_CH,), jnp.float32),
    ]
    + [pltpu.VMEM((_CH,), jnp.int32)] * 8
    + [pltpu.SemaphoreType.DMA] * 12,
    compiler_params=pltpu.CompilerParams(use_tc_tiling_on_sc=False),
)


def _scatter_body(hei, norm, zer, out, S,
                  nb0, nb1, nb2, nb3, eb0, eb1, eb2, eb3, rows0, rows1,
                  sn0, sn1, sn2, sn3, se0, se1, se2, se3,
                  sg0, sg1, ss0, ss1):
  c = lax.axis_index("c")
  s = lax.axis_index("s")
  wid = c * NS + s
  nbufs = [nb0, nb1, nb2, nb3]
  ebufs = [eb0, eb1, eb2, eb3]
  rows = [rows0, rows1]
  sns = [sn0, sn1, sn2, sn3]
  ses = [se0, se1, se2, se3]
  sg = [sg0, sg1]
  ss = [ss0, ss1]

  pltpu.sync_copy(zer, S.at[pl.ds(s * _SR, _SR), :])
  plsc.subcore_barrier()

  q0 = wid * _Q

  def start_idx(i, u):
    pltpu.async_copy(hei.at[0, pl.ds(i * _CH, _CH)], nbufs[u], sns[u])
    pltpu.async_copy(hei.at[1, pl.ds(i * _CH, _CH)], ebufs[u], ses[u])

  for u in range(2):
    start_idx(q0 + u, u)

  def step(t, u):
    i = q0 + t
    p = u % 2
    um = (u + 3) % 4
    pm = (p + 1) % 2

    @pl.when(t >= 2)
    def _():
      pltpu.make_async_copy(rows[p], S.at[ebufs[(u + 2) % 4]], ss[p]).wait()

    @pl.when(t + 2 < _Q)
    def _():
      start_idx(i + 2, (u + 2) % 4)

    pltpu.make_async_copy(hei.at[0, pl.ds(i * _CH, _CH)], nbufs[u], sns[u]).wait()
    pltpu.async_copy(norm.at[nbufs[u]], rows[p], sg[p])

    @pl.when(t >= 1)
    def _():
      pltpu.make_async_copy(norm.at[nbufs[um]], rows[pm], sg[pm]).wait()
      pltpu.make_async_copy(
          hei.at[1, pl.ds((i - 1) * _CH, _CH)], ebufs[um], ses[um]
      ).wait()
      pltpu.async_copy(rows[pm], S.at[ebufs[um]], ss[pm], add=True)

  def outer(m, carry):
    for j in range(4):
      step(m * 4 + j, j)
    return carry

  lax.fori_loop(0, _Q // 4, outer, 0)
  for j in range(_Q % 4):
    step(jnp.int32((_Q // 4) * 4 + j), j)

  uL = (_Q - 1) % 4
  pL = uL % 2
  pltpu.make_async_copy(norm.at[nbufs[uL]], rows[pL], sg[pL]).wait()
  pltpu.make_async_copy(
      hei.at[1, pl.ds((q0 + _Q - 1) * _CH, _CH)], ebufs[uL], ses[uL]
  ).wait()
  pltpu.async_copy(rows[pL], S.at[ebufs[uL]], ss[pL], add=True)
  pltpu.make_async_copy(rows[0], S.at[ebufs[0]], ss[0]).wait()
  pltpu.make_async_copy(rows[1], S.at[ebufs[1]], ss[1]).wait()

  plsc.subcore_barrier()
  pltpu.sync_copy(
      S.at[pl.ds(s * _SR, _SR), :], out.at[c, pl.ds(s * _SR, _SR), :]
  )


_scatter_call = pl.kernel(
    _scatter_body,
    out_type=jax.ShapeDtypeStruct((NC, _NP, _K), jnp.float32),
    mesh=_mesh,
    scratch_types=[
        pltpu.VMEM_SHARED((_NP, _K), jnp.float32),
    ]
    + [pltpu.VMEM((_CH,), jnp.int32)] * 8
    + [pltpu.VMEM((_CH, _K), jnp.float32)] * 2
    + [pltpu.SemaphoreType.DMA] * 12,
    compiler_params=pltpu.CompilerParams(use_tc_tiling_on_sc=False),
)

_NR = _N // 16
_NR2 = _NP // 16


def _lane_repeat_mat():
  li = lax.broadcasted_iota(jnp.int32, (16, 128), 1)
  ri = lax.broadcasted_iota(jnp.int32, (16, 128), 0)
  return (li // 8 == ri).astype(jnp.float32)


def _lane_fold_mat():
  li = lax.broadcasted_iota(jnp.int32, (128, 8), 0)
  ci = lax.broadcasted_iota(jnp.int32, (128, 8), 1)
  return (li % 8 == ci).astype(jnp.float32)


_HR = _NH // 16


def _norm_body(hist2, z2, norm_o, dvc_o, dei_o):
  dv = hist2[0 * _HR : 0 * _HR + _NR] + hist2[2 * _HR : 2 * _HR + _NR]
  de = hist2[1 * _HR : 1 * _HR + _NR] + hist2[3 * _HR : 3 * _HR + _NR]
  dvc = jnp.maximum(dv, 1e-06)
  dec = jnp.maximum(de, 1e-06)
  dvc_o[...] = dvc
  dei_o[...] = 1.0 / dec
  dvis = lax.rsqrt(dvc)
  rep = jnp.dot(dvis, _lane_repeat_mat(), preferred_element_type=jnp.float32)
  norm_o[...] = z2[...] * rep


_norm_call = pl.pallas_call(
    _norm_body,
    out_shape=[
        jax.ShapeDtypeStruct((_NR, 128), jnp.float32),
        jax.ShapeDtypeStruct((_NR, 16), jnp.float32),
        jax.ShapeDtypeStruct((_NR, 16), jnp.float32),
    ],
)


def _final_body(sp2, z2, dvc, dei, out):
  s2 = sp2[0, : _NR] + sp2[1, : _NR]
  rep = _lane_repeat_mat()
  dei_rep = jnp.dot(dei[...], rep, preferred_element_type=jnp.float32)
  dvc_rep = jnp.dot(dvc[...], rep, preferred_element_type=jnp.float32)
  z = z2[...]
  t128 = jnp.sum(s2 * s2 * dei_rep, axis=0, keepdims=True)
  f128 = jnp.sum(z * z * dvc_rep, axis=0, keepdims=True)
  fold = _lane_fold_mat()
  theta = jnp.dot(t128, fold, preferred_element_type=jnp.float32)
  fdvf = jnp.maximum(
      jnp.dot(f128, fold, preferred_element_type=jnp.float32), 1e-06
  )
  rq = 1.0 - theta / fdvf
  rq = jnp.where(jnp.isnan(rq) | jnp.isinf(rq), 0.0, rq)
  rayleigh = jnp.sum(rq) / _K
  colmask = lax.broadcasted_iota(jnp.int32, (_NR, 128), 1) % 8 == 0
  posm = colmask & (z > 0.0)
  negm = colmask & (z < 0.0)
  ps = jnp.sum(jnp.where(posm, z, 0.0))
  ns = jnp.sum(jnp.where(negm, z, 0.0))
  pc = jnp.sum(posm.astype(jnp.float32))
  nc = jnp.sum(negm.astype(jnp.float32))
  pos_mean = ps / jnp.maximum(pc, 1.0)
  neg_mean = ns / jnp.maximum(nc, 1.0)
  separation = jnp.abs(pos_mean - neg_mean)
  penalty = jnp.where((pc == 0.0) | (nc == 0.0), 1.0, 1.0 / (separation + 1e-06))
  out[0, 0] = rayleigh + 0.1 * penalty


_final_call = pl.pallas_call(
    _final_body,
    out_specs=pl.BlockSpec(memory_space=pltpu.SMEM),
    out_shape=jax.ShapeDtypeStruct((1, 1), jnp.float32),
)


@jax.jit
def _run(Z, hyperedge_index):
  hist2 = _hist_call(hyperedge_index).reshape(NC * 2 * _NH // 16, 16)
  z2 = Z.reshape(_NR, 128)
  norm2, dvc, dei = _norm_call(hist2, z2)
  norm = norm2.reshape(_N, _K)
  zeros_nk = jnp.zeros((_SR, _K), jnp.float32)
  sp = _scatter_call(hyperedge_index, norm, zeros_nk)
  sp2 = sp.reshape(NC, _NR2, 128)
  out = _final_call(sp2, z2, dvc, dei)
  return out[0, 0]


def kernel(Z, hyperedge_index, num_nodes):
  return _run(Z, hyperedge_index) + 0.0 * jnp.asarray(num_nodes, jnp.float32)


# speedup vs baseline: 101.9771x; 1.0003x over previous
import jax
import jax.numpy as jnp
from jax import lax
from jax.experimental import pallas as pl
from jax.experimental.pallas import tpu as pltpu
from jax.experimental.pallas import tpu_sc as plsc

NC = 2
NS = 16
NW = NC * NS

_N = 100000
_K = 8
_E = 3200000
_CH = 2000
_Q = _E // _CH // NW
_NGRP = _E // _CH
_NH = 102400
_HS = _NH // NS
_NP = 100096
_SR = _NP // NS
_NSR = _N // NS

_mesh = plsc.VectorSubcoreMesh(
    core_axis_name="c", subcore_axis_name="s", num_cores=NC, num_subcores=NS
)


def _grp_range(wid):
  g0 = (_NGRP * wid) // NW
  g1 = (_NGRP * (wid + 1)) // NW
  return g0, g1


def _hist_body(hei, out, hv, he, zbuf, onesb,
               nb0, nb1, nb2, nb3, eb0, eb1, eb2, eb3,
               sn0, sn1, sn2, sn3, se0, se1, se2, se3,
               shv0, shv1, she0, she1):
  c = lax.axis_index("c")
  s = lax.axis_index("s")
  wid = c * NS + s
  nbufs = [nb0, nb1, nb2, nb3]
  ebufs = [eb0, eb1, eb2, eb3]
  sns = [sn0, sn1, sn2, sn3]
  ses = [se0, se1, se2, se3]
  shv = [shv0, shv1]
  she = [she0, she1]

  def zero_zbuf(i, carry):
    zbuf[pl.ds(i * 16, 16)] = jnp.zeros((16,), jnp.float32)
    return carry

  lax.fori_loop(0, _HS // 16, zero_zbuf, 0)

  def fill_ones(i, carry):
    onesb[pl.ds(i * 16, 16)] = jnp.ones((16,), jnp.float32)
    return carry

  lax.fori_loop(0, _CH // 16, fill_ones, 0)
  pltpu.sync_copy(zbuf, hv.at[pl.ds(s * _HS, _HS)])
  pltpu.sync_copy(zbuf, he.at[pl.ds(s * _HS, _HS)])
  plsc.subcore_barrier()

  q0, q1 = _grp_range(wid)
  n = q1 - q0

  def start_idx(i, u):
    pltpu.async_copy(hei.at[pl.ds(i * _CH, _CH)], nbufs[u], sns[u])
    pltpu.async_copy(hei.at[pl.ds(_E + i * _CH, _CH)], ebufs[u], ses[u])

  for u in range(2):
    @pl.when(q0 + u < q1)
    def _():
      start_idx(q0 + u, u)

  def outer(m, carry):
    for u in range(4):
      i = q0 + m * 4 + u

      @pl.when(i < q1)
      def _():
        p = u % 2

        @pl.when(i - 2 >= q0)
        def _():
          pltpu.make_async_copy(onesb, hv.at[nbufs[(u + 2) % 4]], shv[p]).wait()
          pltpu.make_async_copy(onesb, he.at[ebufs[(u + 2) % 4]], she[p]).wait()

        @pl.when(i + 2 < q1)
        def _():
          start_idx(i + 2, (u + 2) % 4)

        pltpu.make_async_copy(hei.at[pl.ds(i * _CH, _CH)], nbufs[u], sns[u]).wait()
        pltpu.make_async_copy(hei.at[pl.ds(_E + i * _CH, _CH)], ebufs[u], ses[u]).wait()
        pltpu.async_copy(onesb, hv.at[nbufs[u]], shv[p], add=True)
        pltpu.async_copy(onesb, he.at[ebufs[u]], she[p], add=True)

    return carry

  lax.fori_loop(0, (n + 3) // 4, outer, 0)
  for p in range(2):
    @pl.when(n > p)
    def _():
      pltpu.make_async_copy(onesb, hv.at[nbufs[p]], shv[p]).wait()
      pltpu.make_async_copy(onesb, he.at[ebufs[p]], she[p]).wait()

  plsc.subcore_barrier()
  base = c * (2 * _NH)
  pltpu.sync_copy(hv.at[pl.ds(s * _HS, _HS)], out.at[pl.ds(base + s * _HS, _HS)])
  pltpu.sync_copy(
      he.at[pl.ds(s * _HS, _HS)], out.at[pl.ds(base + _NH + s * _HS, _HS)]
  )


_hist_call = pl.kernel(
    _hist_body,
    out_type=jax.ShapeDtypeStruct((NC * 2 * _NH,), jnp.float32),
    mesh=_mesh,
    scratch_types=[
        pltpu.VMEM_SHARED((_NH,), jnp.float32),
        pltpu.VMEM_SHARED((_NH,), jnp.float32),
        pltpu.VMEM((_HS,), jnp.float32),
        pltpu.VMEM((_CH,), jnp.float32),
    ]
    + [pltpu.VMEM((_CH,), jnp.int32)] * 8
    + [pltpu.SemaphoreType.DMA] * 12,
    compiler_params=pltpu.CompilerParams(use_tc_tiling_on_sc=False),
)


def _scatter_body(hei, norm, zer, out, S,
                  nb0, nb1, nb2, nb3, eb0, eb1, eb2, eb3, rows0, rows1,
                  sn0, sn1, sn2, sn3, se0, se1, se2, se3,
                  sg0, sg1, ss0, ss1):
  c = lax.axis_index("c")
  s = lax.axis_index("s")
  wid = c * NS + s
  nbufs = [nb0, nb1, nb2, nb3]
  ebufs = [eb0, eb1, eb2, eb3]
  rows = [rows0, rows1]
  sns = [sn0, sn1, sn2, sn3]
  ses = [se0, se1, se2, se3]
  sg = [sg0, sg1]
  ss = [ss0, ss1]

  pltpu.sync_copy(zer, S.at[pl.ds(s * _SR, _SR), :])
  plsc.subcore_barrier()

  q0 = wid * _Q

  def start_idx(i, u):
    pltpu.async_copy(hei.at[pl.ds(i * _CH, _CH)], nbufs[u], sns[u])
    pltpu.async_copy(hei.at[pl.ds(_E + i * _CH, _CH)], ebufs[u], ses[u])

  for u in range(2):
    start_idx(q0 + u, u)

  def step(t, u):
    i = q0 + t
    p = u % 2
    um = (u + 3) % 4
    pm = (p + 1) % 2

    @pl.when(t >= 2)
    def _():
      pltpu.make_async_copy(rows[p], S.at[ebufs[(u + 2) % 4]], ss[p]).wait()

    @pl.when(t + 2 < _Q)
    def _():
      start_idx(i + 2, (u + 2) % 4)

    pltpu.make_async_copy(hei.at[pl.ds(i * _CH, _CH)], nbufs[u], sns[u]).wait()
    pltpu.async_copy(norm.at[nbufs[u]], rows[p], sg[p])

    @pl.when(t >= 1)
    def _():
      pltpu.make_async_copy(norm.at[nbufs[um]], rows[pm], sg[pm]).wait()
      pltpu.make_async_copy(
          hei.at[pl.ds(_E + (i - 1) * _CH, _CH)], ebufs[um], ses[um]
      ).wait()
      pltpu.async_copy(rows[pm], S.at[ebufs[um]], ss[pm], add=True)

  def outer(m, carry):
    for j in range(4):
      step(m * 4 + j, j)
    return carry

  lax.fori_loop(0, _Q // 4, outer, 0)
  for j in range(_Q % 4):
    step(jnp.int32((_Q // 4) * 4 + j), j)

  uL = (_Q - 1) % 4
  pL = uL % 2
  pltpu.make_async_copy(norm.at[nbufs[uL]], rows[pL], sg[pL]).wait()
  pltpu.make_async_copy(
      hei.at[pl.ds(_E + (q0 + _Q - 1) * _CH, _CH)], ebufs[uL], ses[uL]
  ).wait()
  pltpu.async_copy(rows[pL], S.at[ebufs[uL]], ss[pL], add=True)
  pltpu.make_async_copy(rows[0], S.at[ebufs[0]], ss[0]).wait()
  pltpu.make_async_copy(rows[1], S.at[ebufs[1]], ss[1]).wait()

  plsc.subcore_barrier()
  pltpu.sync_copy(
      S.at[pl.ds(s * _SR, _SR), :], out.at[c, pl.ds(s * _SR, _SR), :]
  )


_scatter_call = pl.kernel(
    _scatter_body,
    out_type=jax.ShapeDtypeStruct((NC, _NP, _K), jnp.float32),
    mesh=_mesh,
    scratch_types=[
        pltpu.VMEM_SHARED((_NP, _K), jnp.float32),
    ]
    + [pltpu.VMEM((_CH,), jnp.int32)] * 8
    + [pltpu.VMEM((_CH, _K), jnp.float32)] * 2
    + [pltpu.SemaphoreType.DMA] * 12,
    compiler_params=pltpu.CompilerParams(use_tc_tiling_on_sc=False),
)

_NR = _N // 16
_NR2 = _NP // 16


def _lane_repeat_mat():
  li = lax.broadcasted_iota(jnp.int32, (16, 128), 1)
  ri = lax.broadcasted_iota(jnp.int32, (16, 128), 0)
  return (li // 8 == ri).astype(jnp.float32)


def _lane_fold_mat():
  li = lax.broadcasted_iota(jnp.int32, (128, 8), 0)
  ci = lax.broadcasted_iota(jnp.int32, (128, 8), 1)
  return (li % 8 == ci).astype(jnp.float32)


_HR = _NH // 16


def _norm_body(hist2, z2, norm_o, dvc_o, dei_o):
  dv = hist2[0 * _HR : 0 * _HR + _NR] + hist2[2 * _HR : 2 * _HR + _NR]
  de = hist2[1 * _HR : 1 * _HR + _NR] + hist2[3 * _HR : 3 * _HR + _NR]
  dvc = jnp.maximum(dv, 1e-06)
  dec = jnp.maximum(de, 1e-06)
  dvc_o[...] = dvc
  dei_o[...] = 1.0 / dec
  dvis = lax.rsqrt(dvc)
  rep = jnp.dot(dvis, _lane_repeat_mat(), preferred_element_type=jnp.float32)
  norm_o[...] = z2[...] * rep


_norm_call = pl.pallas_call(
    _norm_body,
    out_shape=[
        jax.ShapeDtypeStruct((_NR, 128), jnp.float32),
        jax.ShapeDtypeStruct((_NR, 16), jnp.float32),
        jax.ShapeDtypeStruct((_NR, 16), jnp.float32),
    ],
)


def _final_body(sp2, z2, dvc, dei, out):
  s2 = sp2[0, : _NR] + sp2[1, : _NR]
  rep = _lane_repeat_mat()
  dei_rep = jnp.dot(dei[...], rep, preferred_element_type=jnp.float32)
  dvc_rep = jnp.dot(dvc[...], rep, preferred_element_type=jnp.float32)
  z = z2[...]
  t128 = jnp.sum(s2 * s2 * dei_rep, axis=0, keepdims=True)
  f128 = jnp.sum(z * z * dvc_rep, axis=0, keepdims=True)
  fold = _lane_fold_mat()
  theta = jnp.dot(t128, fold, preferred_element_type=jnp.float32)
  fdvf = jnp.maximum(
      jnp.dot(f128, fold, preferred_element_type=jnp.float32), 1e-06
  )
  rq = 1.0 - theta / fdvf
  rq = jnp.where(jnp.isnan(rq) | jnp.isinf(rq), 0.0, rq)
  rayleigh = jnp.sum(rq) / _K
  colmask = lax.broadcasted_iota(jnp.int32, (_NR, 128), 1) % 8 == 0
  posm = colmask & (z > 0.0)
  negm = colmask & (z < 0.0)
  ps = jnp.sum(jnp.where(posm, z, 0.0))
  ns = jnp.sum(jnp.where(negm, z, 0.0))
  pc = jnp.sum(posm.astype(jnp.float32))
  nc = jnp.sum(negm.astype(jnp.float32))
  pos_mean = ps / jnp.maximum(pc, 1.0)
  neg_mean = ns / jnp.maximum(nc, 1.0)
  separation = jnp.abs(pos_mean - neg_mean)
  penalty = jnp.where((pc == 0.0) | (nc == 0.0), 1.0, 1.0 / (separation + 1e-06))
  out[0, 0] = rayleigh + 0.1 * penalty


_final_call = pl.pallas_call(
    _final_body,
    out_specs=pl.BlockSpec(memory_space=pltpu.SMEM),
    out_shape=jax.ShapeDtypeStruct((1, 1), jnp.float32),
)


@jax.jit
def _run(Z, hyperedge_index):
  hei1 = hyperedge_index.reshape(2 * _E)
  hist2 = _hist_call(hei1).reshape(NC * 2 * _NH // 16, 16)
  z2 = Z.reshape(_NR, 128)
  norm2, dvc, dei = _norm_call(hist2, z2)
  norm = norm2.reshape(_N, _K)
  zeros_nk = jnp.zeros((_SR, _K), jnp.float32)
  sp = _scatter_call(hei1, norm, zeros_nk)
  sp2 = sp.reshape(NC, _NR2, 128)
  out = _final_call(sp2, z2, dvc, dei)
  return out[0, 0]


def kernel(Z, hyperedge_index, num_nodes):
  return _run(Z, hyperedge_index) + 0.0 * jnp.asarray(num_nodes, jnp.float32)
